# baseline (speedup 1.0000x reference)
import functools

import jax
import jax.numpy as jnp
from jax import lax
from jax.experimental import pallas as pl
from jax.experimental.pallas import tpu as pltpu
from jax.experimental.pallas import tpu_sc as plsc

N = 10000
CH = 256
K = 27
TN = 256
NT = (N + TN - 1) // TN
NPAD = NT * TN
GW = 64
NWORK = 32
NBUF = 4
ROWS = K * NPAD
_CHUNK = GW * NWORK * NBUF
L = ((ROWS + _CHUNK - 1) // _CHUNK) * _CHUNK
PER = L // NWORK
NWIN = PER // GW
assert NWIN % NBUF == 0


def _sc_gather(xpad, idx_flat):
    width = xpad.shape[1]

    @functools.partial(
        pl.kernel,
        out_type=jax.ShapeDtypeStruct((L, width), xpad.dtype),
        mesh=plsc.VectorSubcoreMesh(core_axis_name="c", subcore_axis_name="s"),
        scratch_types=[
            pltpu.VMEM((PER,), jnp.int32),
            pltpu.VMEM((NBUF, GW, width), xpad.dtype),
            pltpu.SemaphoreType.DMA((NBUF,)),
            pltpu.SemaphoreType.DMA((NBUF,)),
            pltpu.SemaphoreType.DMA,
        ],
    )
    def kern(x_hbm, i_hbm, o_hbm, idx_v, rows_v, gsem, osem, isem):
        wid = lax.axis_index("s") * 2 + lax.axis_index("c")
        base = wid * PER
        pltpu.async_copy(i_hbm.at[pl.ds(base, PER)], idx_v, isem).wait()

        def gather_start(w, b):
            pltpu.make_async_copy(
                x_hbm.at[idx_v.at[pl.ds(w * GW, GW)]],
                rows_v.at[b], gsem.at[b]).start()

        def write_start(w, b):
            pltpu.make_async_copy(
                rows_v.at[b],
                o_hbm.at[pl.ds(base + w * GW, GW)], osem.at[b]).start()

        def gather_wait(b):
            pltpu.make_async_copy(
                x_hbm.at[idx_v.at[pl.ds(0, GW)]],
                rows_v.at[b], gsem.at[b]).wait()

        def write_wait(w, b):
            pltpu.make_async_copy(
                rows_v.at[b],
                o_hbm.at[pl.ds(base + w * GW, GW)], osem.at[b]).wait()

        for b in range(NBUF):
            gather_start(b, b)

        @pl.loop(0, NWIN, step=NBUF)
        def _(g):
            for b in range(NBUF):
                w = g + b
                gather_wait(b)
                write_start(w, b)

                @pl.when(w + NBUF < NWIN)
                def _():
                    write_wait(w, b)
                    gather_start(w + NBUF, b)

        for b in range(NBUF):
            write_wait(NWIN - NBUF + b, b)

    return kern(xpad, idx_flat)


def _tc_conv(g, w, b, a, res, out_shape, out_dtype, mask_tail):
    residual = res is not None

    def body(*refs):
        if residual:
            g_ref, w_ref, b_ref, a_ref, res_ref, o_ref, acc_ref = refs
        else:
            g_ref, w_ref, b_ref, a_ref, o_ref, acc_ref = refs
        k = pl.program_id(1)

        @pl.when(k == 0)
        def _():
            acc_ref[...] = jnp.zeros_like(acc_ref)

        acc_ref[...] += jnp.dot(g_ref[...].astype(jnp.bfloat16), w_ref[k],
                                preferred_element_type=jnp.float32)

        @pl.when(k == K - 1)
        def _():
            x = acc_ref[...] + b_ref[...]
            if residual:
                x = x + res_ref[...]
            x = jnp.where(x >= 0, x, a_ref[0, 0] * x)
            if mask_tail:
                nt = pl.program_id(0)
                rows = nt * TN + jax.lax.broadcasted_iota(
                    jnp.int32, (TN, CH), 0)
                x = jnp.where(rows < N, x, 0.0)
            o_ref[...] = x.astype(o_ref.dtype)

    in_specs = [
        pl.BlockSpec((TN, CH), lambda nt, k: (k * NT + nt, 0)),
        pl.BlockSpec((K, CH, CH), lambda nt, k: (0, 0, 0)),
        pl.BlockSpec((1, CH), lambda nt, k: (0, 0)),
        pl.BlockSpec((1, 1), lambda nt, k: (0, 0)),
    ]
    args = [g, w, b.reshape(1, CH), a.reshape(1, 1)]
    if residual:
        in_specs.append(pl.BlockSpec((TN, CH), lambda nt, k: (nt, 0)))
        args.append(res)

    return pl.pallas_call(
        body,
        grid=(NT, K),
        in_specs=in_specs,
        out_specs=pl.BlockSpec((TN, CH), lambda nt, k: (nt, 0)),
        out_shape=jax.ShapeDtypeStruct(out_shape, out_dtype),
        scratch_shapes=[pltpu.VMEM((TN, CH), jnp.float32)],
        compiler_params=pltpu.CompilerParams(
            dimension_semantics=("parallel", "arbitrary")),
    )(*args)


def kernel(feats, nbr_idx, W1, b1, a1, W2, b2, a2):
    idxT = nbr_idx.T.astype(jnp.int32)
    idx_pad = jnp.full((K, NPAD), N, jnp.int32).at[:, :N].set(idxT)
    flat0 = jnp.full((L,), N, jnp.int32).at[:ROWS].set(
        idx_pad.reshape(-1))
    pos = jnp.arange(L, dtype=jnp.int32)
    idx_flat = jnp.where(flat0 == N, N + pos % (NPAD - N), flat0)

    w1b = W1.astype(jnp.bfloat16)
    w2b = W2.astype(jnp.bfloat16)

    def pack(x):
        return lax.bitcast_convert_type(
            x.reshape(x.shape[0], CH // 2, 2), jnp.float32)

    def unpack(x):
        return lax.bitcast_convert_type(x, jnp.bfloat16).reshape(-1, CH)

    fb = feats.astype(jnp.bfloat16)
    xpad0 = jnp.concatenate(
        [fb, jnp.zeros((NPAD - N, CH), jnp.bfloat16)], axis=0)
    g1 = unpack(_sc_gather(pack(xpad0), idx_flat))
    x1 = _tc_conv(g1, w1b, b1, a1, None, (NPAD, CH), jnp.bfloat16,
                  mask_tail=True)

    g2 = unpack(_sc_gather(pack(x1), idx_flat))
    out = _tc_conv(g2, w2b, b2, a2, feats, (N, CH), jnp.float32,
                   mask_tail=False)
    return out


# speedup vs baseline: 2.3235x; 2.3235x over previous
import functools

import numpy as np
import jax
import jax.numpy as jnp
from jax import lax
from jax.experimental import pallas as pl
from jax.experimental.pallas import tpu as pltpu
from jax.experimental.pallas import tpu_sc as plsc

N = 10000
CH = 256
K = 27
TN = 256
NT = (N + TN - 1) // TN
NPAD = NT * TN
GW = 64
NWORK = 32
NBUF = 4
ROWS = K * NPAD
_CHUNK = GW * NWORK * NBUF
L = ((ROWS + _CHUNK - 1) // _CHUNK) * _CHUNK
PER = L // NWORK
NWIN = PER // GW
assert NWIN % NBUF == 0


def _sc_gather(xpad, idx_flat):
    width = xpad.shape[1]

    @functools.partial(
        pl.kernel,
        out_type=jax.ShapeDtypeStruct((L, width), xpad.dtype),
        mesh=plsc.VectorSubcoreMesh(core_axis_name="c", subcore_axis_name="s"),
        scratch_types=[
            pltpu.VMEM((PER,), jnp.int32),
            pltpu.VMEM((NBUF, GW, width), xpad.dtype),
            pltpu.SemaphoreType.DMA((NBUF,)),
            pltpu.SemaphoreType.DMA((NBUF,)),
            pltpu.SemaphoreType.DMA,
        ],
    )
    def kern(x_hbm, i_hbm, o_hbm, idx_v, rows_v, gsem, osem, isem):
        wid = lax.axis_index("s") * 2 + lax.axis_index("c")
        base = wid * PER
        pltpu.async_copy(i_hbm.at[pl.ds(base, PER)], idx_v, isem).wait()

        def gather_start(w, b):
            pltpu.make_async_copy(
                x_hbm.at[idx_v.at[pl.ds(w * GW, GW)]],
                rows_v.at[b], gsem.at[b]).start()

        def write_start(w, b):
            pltpu.make_async_copy(
                rows_v.at[b],
                o_hbm.at[pl.ds(base + w * GW, GW)], osem.at[b]).start()

        def gather_wait(b):
            pltpu.make_async_copy(
                x_hbm.at[idx_v.at[pl.ds(0, GW)]],
                rows_v.at[b], gsem.at[b]).wait()

        def write_wait(w, b):
            pltpu.make_async_copy(
                rows_v.at[b],
                o_hbm.at[pl.ds(base + w * GW, GW)], osem.at[b]).wait()

        for b in range(NBUF):
            gather_start(b, b)

        @pl.loop(0, NWIN, step=NBUF)
        def _(g):
            for b in range(NBUF):
                w = g + b
                gather_wait(b)
                write_start(w, b)

                @pl.when(w + NBUF < NWIN)
                def _():
                    write_wait(w, b)
                    gather_start(w + NBUF, b)

        for b in range(NBUF):
            write_wait(NWIN - NBUF + b, b)

    return kern(xpad, idx_flat)


HW = CH // 2
_HIMASK = np.uint32(0xFFFF0000)


def _unpack(p):
    u = lax.bitcast_convert_type(p, jnp.uint32)
    lo = lax.bitcast_convert_type(u << 16, jnp.float32)
    hi = lax.bitcast_convert_type(u & _HIMASK, jnp.float32)
    return jnp.concatenate([lo, hi], axis=1).astype(jnp.bfloat16)


def _pack(x):
    b = x.astype(jnp.bfloat16)
    ulo = lax.bitcast_convert_type(
        b[:, :HW].astype(jnp.float32), jnp.uint32)
    uhi = lax.bitcast_convert_type(
        b[:, HW:].astype(jnp.float32), jnp.uint32)
    return lax.bitcast_convert_type((ulo >> 16) | (uhi & _HIMASK),
                                    jnp.float32)


def _tc_conv(g, w, b, a, res, out_shape, out_dtype, mask_tail, pack_out):
    residual = res is not None

    def body(*refs):
        if residual:
            g_ref, w_ref, b_ref, a_ref, res_ref, o_ref, acc_ref = refs
        else:
            g_ref, w_ref, b_ref, a_ref, o_ref, acc_ref = refs
        k = pl.program_id(1)

        @pl.when(k == 0)
        def _():
            acc_ref[...] = jnp.zeros_like(acc_ref)

        acc_ref[...] += jnp.dot(_unpack(g_ref[...]), w_ref[k],
                                preferred_element_type=jnp.float32)

        @pl.when(k == K - 1)
        def _():
            x = acc_ref[...] + b_ref[...]
            if residual:
                x = x + res_ref[...]
            x = jnp.where(x >= 0, x, a_ref[0, 0] * x)
            if mask_tail:
                nt = pl.program_id(0)
                rows = nt * TN + jax.lax.broadcasted_iota(
                    jnp.int32, (TN, CH), 0)
                x = jnp.where(rows < N, x, 0.0)
            if pack_out:
                o_ref[...] = _pack(x)
            else:
                o_ref[...] = x.astype(o_ref.dtype)

    in_specs = [
        pl.BlockSpec((TN, HW), lambda nt, k: (k * NT + nt, 0)),
        pl.BlockSpec((K, CH, CH), lambda nt, k: (0, 0, 0)),
        pl.BlockSpec((1, CH), lambda nt, k: (0, 0)),
        pl.BlockSpec((1, 1), lambda nt, k: (0, 0)),
    ]
    args = [g, w, b.reshape(1, CH), a.reshape(1, 1)]
    if residual:
        in_specs.append(pl.BlockSpec((TN, CH), lambda nt, k: (nt, 0)))
        args.append(res)

    ow = HW if pack_out else CH
    return pl.pallas_call(
        body,
        grid=(NT, K),
        in_specs=in_specs,
        out_specs=pl.BlockSpec((TN, ow), lambda nt, k: (nt, 0)),
        out_shape=jax.ShapeDtypeStruct(out_shape, out_dtype),
        scratch_shapes=[pltpu.VMEM((TN, CH), jnp.float32)],
        compiler_params=pltpu.CompilerParams(
            dimension_semantics=("parallel", "arbitrary")),
    )(*args)


def kernel(feats, nbr_idx, W1, b1, a1, W2, b2, a2):
    idxT = nbr_idx.T.astype(jnp.int32)
    idx_pad = jnp.full((K, NPAD), N, jnp.int32).at[:, :N].set(idxT)
    flat0 = jnp.full((L,), N, jnp.int32).at[:ROWS].set(
        idx_pad.reshape(-1))
    pos = jnp.arange(L, dtype=jnp.int32)
    idx_flat = jnp.where(flat0 == N, N + pos % (NPAD - N), flat0)

    w1b = W1.astype(jnp.bfloat16)
    w2b = W2.astype(jnp.bfloat16)

    fb = feats.astype(jnp.bfloat16)
    ulo = lax.bitcast_convert_type(
        fb[:, :HW].astype(jnp.float32), jnp.uint32)
    uhi = lax.bitcast_convert_type(
        fb[:, HW:].astype(jnp.float32), jnp.uint32)
    fpacked = lax.bitcast_convert_type((ulo >> 16) | (uhi & _HIMASK),
                                       jnp.float32)
    xpad0 = jnp.concatenate(
        [fpacked, jnp.zeros((NPAD - N, HW), jnp.float32)], axis=0)
    g1 = _sc_gather(xpad0, idx_flat)
    x1 = _tc_conv(g1, w1b, b1, a1, None, (NPAD, HW), jnp.float32,
                  mask_tail=True, pack_out=True)

    g2 = _sc_gather(x1, idx_flat)
    out = _tc_conv(g2, w2b, b2, a2, feats, (N, CH), jnp.float32,
                   mask_tail=False, pack_out=False)
    return out


# speedup vs baseline: 3.4081x; 1.4668x over previous
import functools

import numpy as np
import jax
import jax.numpy as jnp
from jax import lax
from jax.experimental import pallas as pl
from jax.experimental.pallas import tpu as pltpu
from jax.experimental.pallas import tpu_sc as plsc

N = 10000
CH = 256
K = 27
TN = 512
NT = (N + TN - 1) // TN
NPAD = NT * TN
GW = 64
NWORK = 32
NBUF = 4
ROWS = K * NPAD
_CHUNK = GW * NWORK * NBUF
L = ((ROWS + _CHUNK - 1) // _CHUNK) * _CHUNK
PER = L // NWORK
NWIN = PER // GW
assert NWIN % NBUF == 0


def _sc_gather(xpad, idx_flat):
    width = xpad.shape[1]

    @functools.partial(
        pl.kernel,
        out_type=jax.ShapeDtypeStruct((L, width), xpad.dtype),
        mesh=plsc.VectorSubcoreMesh(core_axis_name="c", subcore_axis_name="s"),
        scratch_types=[
            pltpu.VMEM((PER,), jnp.int32),
            pltpu.VMEM((NBUF, GW, width), xpad.dtype),
            pltpu.SemaphoreType.DMA((NBUF,)),
            pltpu.SemaphoreType.DMA((NBUF,)),
            pltpu.SemaphoreType.DMA,
        ],
    )
    def kern(x_hbm, i_hbm, o_hbm, idx_v, rows_v, gsem, osem, isem):
        wid = lax.axis_index("s") * 2 + lax.axis_index("c")
        base = wid * PER
        pltpu.async_copy(i_hbm.at[pl.ds(base, PER)], idx_v, isem).wait()

        def gather_start(w, b):
            pltpu.make_async_copy(
                x_hbm.at[idx_v.at[pl.ds(w * GW, GW)]],
                rows_v.at[b], gsem.at[b]).start()

        def write_start(w, b):
            pltpu.make_async_copy(
                rows_v.at[b],
                o_hbm.at[pl.ds(base + w * GW, GW)], osem.at[b]).start()

        def gather_wait(b):
            pltpu.make_async_copy(
                x_hbm.at[idx_v.at[pl.ds(0, GW)]],
                rows_v.at[b], gsem.at[b]).wait()

        def write_wait(w, b):
            pltpu.make_async_copy(
                rows_v.at[b],
                o_hbm.at[pl.ds(base + w * GW, GW)], osem.at[b]).wait()

        for b in range(NBUF):
            gather_start(b, b)

        @pl.loop(0, NWIN, step=NBUF)
        def _(g):
            for b in range(NBUF):
                w = g + b
                gather_wait(b)
                write_start(w, b)

                @pl.when(w + NBUF < NWIN)
                def _():
                    write_wait(w, b)
                    gather_start(w + NBUF, b)

        for b in range(NBUF):
            write_wait(NWIN - NBUF + b, b)

    return kern(xpad, idx_flat)


HW = CH // 2
_HIMASK = np.uint32(0xFFFF0000)


def _unpack(p):
    u = lax.bitcast_convert_type(p, jnp.uint32)
    lo = lax.bitcast_convert_type(u << 16, jnp.float32)
    hi = lax.bitcast_convert_type(u & _HIMASK, jnp.float32)
    return jnp.concatenate([lo, hi], axis=1).astype(jnp.bfloat16)


def _pack(x):
    b = x.astype(jnp.bfloat16)
    ulo = lax.bitcast_convert_type(
        b[:, :HW].astype(jnp.float32), jnp.uint32)
    uhi = lax.bitcast_convert_type(
        b[:, HW:].astype(jnp.float32), jnp.uint32)
    return lax.bitcast_convert_type((ulo >> 16) | (uhi & _HIMASK),
                                    jnp.float32)


def _tc_conv(g, w, b, a, res, out_shape, out_dtype, mask_tail, pack_out):
    residual = res is not None

    def body(*refs):
        if residual:
            g_ref, w_ref, b_ref, a_ref, res_ref, o_ref, acc_ref = refs
        else:
            g_ref, w_ref, b_ref, a_ref, o_ref, acc_ref = refs
        k = pl.program_id(1)

        @pl.when(k == 0)
        def _():
            acc_ref[...] = jnp.zeros_like(acc_ref)

        acc_ref[...] += jnp.dot(_unpack(g_ref[...]), w_ref[k],
                                preferred_element_type=jnp.float32)

        @pl.when(k == K - 1)
        def _():
            x = acc_ref[...] + b_ref[...]
            if residual:
                x = x + res_ref[...]
            x = jnp.where(x >= 0, x, a_ref[0, 0] * x)
            if mask_tail:
                nt = pl.program_id(0)
                rows = nt * TN + jax.lax.broadcasted_iota(
                    jnp.int32, (TN, CH), 0)
                x = jnp.where(rows < N, x, 0.0)
            if pack_out:
                o_ref[...] = _pack(x)
            else:
                o_ref[...] = x.astype(o_ref.dtype)

    in_specs = [
        pl.BlockSpec((TN, HW), lambda nt, k: (k * NT + nt, 0)),
        pl.BlockSpec((K, CH, CH), lambda nt, k: (0, 0, 0)),
        pl.BlockSpec((1, CH), lambda nt, k: (0, 0)),
        pl.BlockSpec((1, 1), lambda nt, k: (0, 0)),
    ]
    args = [g, w, b.reshape(1, CH), a.reshape(1, 1)]
    if residual:
        in_specs.append(pl.BlockSpec((TN, CH), lambda nt, k: (nt, 0)))
        args.append(res)

    ow = HW if pack_out else CH
    return pl.pallas_call(
        body,
        grid=(NT, K),
        in_specs=in_specs,
        out_specs=pl.BlockSpec((TN, ow), lambda nt, k: (nt, 0)),
        out_shape=jax.ShapeDtypeStruct(out_shape, out_dtype),
        scratch_shapes=[pltpu.VMEM((TN, CH), jnp.float32)],
        compiler_params=pltpu.CompilerParams(
            dimension_semantics=("parallel", "arbitrary")),
    )(*args)


def kernel(feats, nbr_idx, W1, b1, a1, W2, b2, a2):
    idxT = nbr_idx.T.astype(jnp.int32)
    idx_pad = jnp.full((K, NPAD), N, jnp.int32).at[:, :N].set(idxT)
    flat0 = jnp.full((L,), N, jnp.int32).at[:ROWS].set(
        idx_pad.reshape(-1))
    pos = jnp.arange(L, dtype=jnp.int32)
    idx_flat = jnp.where(flat0 == N, N + pos % (NPAD - N), flat0)

    w1b = W1.astype(jnp.bfloat16)
    w2b = W2.astype(jnp.bfloat16)

    fb = feats.astype(jnp.bfloat16)
    ulo = lax.bitcast_convert_type(
        fb[:, :HW].astype(jnp.float32), jnp.uint32)
    uhi = lax.bitcast_convert_type(
        fb[:, HW:].astype(jnp.float32), jnp.uint32)
    fpacked = lax.bitcast_convert_type((ulo >> 16) | (uhi & _HIMASK),
                                       jnp.float32)
    xpad0 = jnp.concatenate(
        [fpacked, jnp.zeros((NPAD - N, HW), jnp.float32)], axis=0)
    g1 = _sc_gather(xpad0, idx_flat)
    x1 = _tc_conv(g1, w1b, b1, a1, None, (NPAD, HW), jnp.float32,
                  mask_tail=True, pack_out=True)

    g2 = _sc_gather(x1, idx_flat)
    out = _tc_conv(g2, w2b, b2, a2, feats, (N, CH), jnp.float32,
                   mask_tail=False, pack_out=False)
    return out


# speedup vs baseline: 4.5120x; 1.3239x over previous
import functools

import numpy as np
import jax
import jax.numpy as jnp
from jax import lax
from jax.experimental import pallas as pl
from jax.experimental.pallas import tpu as pltpu
from jax.experimental.pallas import tpu_sc as plsc

N = 10000
CH = 256
K = 27
TN = 1024
NT = (N + TN - 1) // TN
NPAD = NT * TN
GW = 64
NWORK = 32
NBUF = 4
ROWS = K * NPAD
_CHUNK = GW * NWORK * NBUF
L = ((ROWS + _CHUNK - 1) // _CHUNK) * _CHUNK
PER = L // NWORK
NWIN = PER // GW
assert NWIN % NBUF == 0


def _sc_gather(xpad, idx_flat):
    width = xpad.shape[1]

    @functools.partial(
        pl.kernel,
        out_type=jax.ShapeDtypeStruct((L, width), xpad.dtype),
        mesh=plsc.VectorSubcoreMesh(core_axis_name="c", subcore_axis_name="s"),
        scratch_types=[
            pltpu.VMEM((PER,), jnp.int32),
            pltpu.VMEM((NBUF, GW, width), xpad.dtype),
            pltpu.SemaphoreType.DMA((NBUF,)),
            pltpu.SemaphoreType.DMA((NBUF,)),
            pltpu.SemaphoreType.DMA,
        ],
    )
    def kern(x_hbm, i_hbm, o_hbm, idx_v, rows_v, gsem, osem, isem):
        wid = lax.axis_index("s") * 2 + lax.axis_index("c")
        base = wid * PER
        pltpu.async_copy(i_hbm.at[pl.ds(base, PER)], idx_v, isem).wait()

        def gather_start(w, b):
            pltpu.make_async_copy(
                x_hbm.at[idx_v.at[pl.ds(w * GW, GW)]],
                rows_v.at[b], gsem.at[b]).start()

        def write_start(w, b):
            pltpu.make_async_copy(
                rows_v.at[b],
                o_hbm.at[pl.ds(base + w * GW, GW)], osem.at[b]).start()

        def gather_wait(b):
            pltpu.make_async_copy(
                x_hbm.at[idx_v.at[pl.ds(0, GW)]],
                rows_v.at[b], gsem.at[b]).wait()

        def write_wait(w, b):
            pltpu.make_async_copy(
                rows_v.at[b],
                o_hbm.at[pl.ds(base + w * GW, GW)], osem.at[b]).wait()

        for b in range(NBUF):
            gather_start(b, b)

        @pl.loop(0, NWIN, step=NBUF)
        def _(g):
            for b in range(NBUF):
                w = g + b
                gather_wait(b)
                write_start(w, b)

                @pl.when(w + NBUF < NWIN)
                def _():
                    write_wait(w, b)
                    gather_start(w + NBUF, b)

        for b in range(NBUF):
            write_wait(NWIN - NBUF + b, b)

    return kern(xpad, idx_flat)


HW = CH // 2
_HIMASK = np.uint32(0xFFFF0000)


def _unpack(p):
    u = lax.bitcast_convert_type(p, jnp.uint32)
    lo = lax.bitcast_convert_type(u << 16, jnp.float32)
    hi = lax.bitcast_convert_type(u & _HIMASK, jnp.float32)
    return jnp.concatenate([lo, hi], axis=1).astype(jnp.bfloat16)


def _pack(x):
    b = x.astype(jnp.bfloat16)
    ulo = lax.bitcast_convert_type(
        b[:, :HW].astype(jnp.float32), jnp.uint32)
    uhi = lax.bitcast_convert_type(
        b[:, HW:].astype(jnp.float32), jnp.uint32)
    return lax.bitcast_convert_type((ulo >> 16) | (uhi & _HIMASK),
                                    jnp.float32)


def _tc_conv(g, w, b, a, res, out_shape, out_dtype, mask_tail, pack_out):
    residual = res is not None

    def body(*refs):
        if residual:
            g_ref, w_ref, b_ref, a_ref, res_ref, o_ref, acc_ref = refs
        else:
            g_ref, w_ref, b_ref, a_ref, o_ref, acc_ref = refs
        k = pl.program_id(1)

        @pl.when(k == 0)
        def _():
            acc_ref[...] = jnp.zeros_like(acc_ref)

        acc_ref[...] += jnp.dot(_unpack(g_ref[...]), w_ref[k],
                                preferred_element_type=jnp.float32)

        @pl.when(k == K - 1)
        def _():
            x = acc_ref[...] + b_ref[...]
            if residual:
                x = x + res_ref[...]
            x = jnp.where(x >= 0, x, a_ref[0, 0] * x)
            if mask_tail:
                nt = pl.program_id(0)
                rows = nt * TN + jax.lax.broadcasted_iota(
                    jnp.int32, (TN, CH), 0)
                x = jnp.where(rows < N, x, 0.0)
            if pack_out:
                o_ref[...] = _pack(x)
            else:
                o_ref[...] = x.astype(o_ref.dtype)

    in_specs = [
        pl.BlockSpec((TN, HW), lambda nt, k: (k * NT + nt, 0)),
        pl.BlockSpec((K, CH, CH), lambda nt, k: (0, 0, 0)),
        pl.BlockSpec((1, CH), lambda nt, k: (0, 0)),
        pl.BlockSpec((1, 1), lambda nt, k: (0, 0)),
    ]
    args = [g, w, b.reshape(1, CH), a.reshape(1, 1)]
    if residual:
        in_specs.append(pl.BlockSpec((TN, CH), lambda nt, k: (nt, 0)))
        args.append(res)

    ow = HW if pack_out else CH
    return pl.pallas_call(
        body,
        grid=(NT, K),
        in_specs=in_specs,
        out_specs=pl.BlockSpec((TN, ow), lambda nt, k: (nt, 0)),
        out_shape=jax.ShapeDtypeStruct(out_shape, out_dtype),
        scratch_shapes=[pltpu.VMEM((TN, CH), jnp.float32)],
        compiler_params=pltpu.CompilerParams(
            dimension_semantics=("parallel", "arbitrary")),
    )(*args)


def kernel(feats, nbr_idx, W1, b1, a1, W2, b2, a2):
    idxT = nbr_idx.T.astype(jnp.int32)
    idx_pad = jnp.full((K, NPAD), N, jnp.int32).at[:, :N].set(idxT)
    flat0 = jnp.full((L,), N, jnp.int32).at[:ROWS].set(
        idx_pad.reshape(-1))
    pos = jnp.arange(L, dtype=jnp.int32)
    idx_flat = jnp.where(flat0 == N, N + pos % (NPAD - N), flat0)

    w1b = W1.astype(jnp.bfloat16)
    w2b = W2.astype(jnp.bfloat16)

    fb = feats.astype(jnp.bfloat16)
    ulo = lax.bitcast_convert_type(
        fb[:, :HW].astype(jnp.float32), jnp.uint32)
    uhi = lax.bitcast_convert_type(
        fb[:, HW:].astype(jnp.float32), jnp.uint32)
    fpacked = lax.bitcast_convert_type((ulo >> 16) | (uhi & _HIMASK),
                                       jnp.float32)
    xpad0 = jnp.concatenate(
        [fpacked, jnp.zeros((NPAD - N, HW), jnp.float32)], axis=0)
    g1 = _sc_gather(xpad0, idx_flat)
    x1 = _tc_conv(g1, w1b, b1, a1, None, (NPAD, HW), jnp.float32,
                  mask_tail=True, pack_out=True)

    g2 = _sc_gather(x1, idx_flat)
    out = _tc_conv(g2, w2b, b2, a2, feats, (N, CH), jnp.float32,
                   mask_tail=False, pack_out=False)
    return out


# speedup vs baseline: 5.4140x; 1.1999x over previous
import functools

import numpy as np
import jax
import jax.numpy as jnp
from jax import lax
from jax.experimental import pallas as pl
from jax.experimental.pallas import tpu as pltpu
from jax.experimental.pallas import tpu_sc as plsc

N = 10000
CH = 256
K = 27
TN = 2048
NT = (N + TN - 1) // TN
NPAD = NT * TN
GW = 64
NWORK = 32
NBUF = 4
ROWS = K * NPAD
_CHUNK = GW * NWORK * NBUF
L = ((ROWS + _CHUNK - 1) // _CHUNK) * _CHUNK
PER = L // NWORK
NWIN = PER // GW
assert NWIN % NBUF == 0


def _sc_gather(xpad, idx_flat):
    width = xpad.shape[1]

    @functools.partial(
        pl.kernel,
        out_type=jax.ShapeDtypeStruct((L, width), xpad.dtype),
        mesh=plsc.VectorSubcoreMesh(core_axis_name="c", subcore_axis_name="s"),
        scratch_types=[
            pltpu.VMEM((PER,), jnp.int32),
            pltpu.VMEM((NBUF, GW, width), xpad.dtype),
            pltpu.SemaphoreType.DMA((NBUF,)),
            pltpu.SemaphoreType.DMA((NBUF,)),
            pltpu.SemaphoreType.DMA,
        ],
    )
    def kern(x_hbm, i_hbm, o_hbm, idx_v, rows_v, gsem, osem, isem):
        wid = lax.axis_index("s") * 2 + lax.axis_index("c")
        base = wid * PER
        pltpu.async_copy(i_hbm.at[pl.ds(base, PER)], idx_v, isem).wait()

        def gather_start(w, b):
            pltpu.make_async_copy(
                x_hbm.at[idx_v.at[pl.ds(w * GW, GW)]],
                rows_v.at[b], gsem.at[b]).start()

        def write_start(w, b):
            pltpu.make_async_copy(
                rows_v.at[b],
                o_hbm.at[pl.ds(base + w * GW, GW)], osem.at[b]).start()

        def gather_wait(b):
            pltpu.make_async_copy(
                x_hbm.at[idx_v.at[pl.ds(0, GW)]],
                rows_v.at[b], gsem.at[b]).wait()

        def write_wait(w, b):
            pltpu.make_async_copy(
                rows_v.at[b],
                o_hbm.at[pl.ds(base + w * GW, GW)], osem.at[b]).wait()

        for b in range(NBUF):
            gather_start(b, b)

        @pl.loop(0, NWIN, step=NBUF)
        def _(g):
            for b in range(NBUF):
                w = g + b
                gather_wait(b)
                write_start(w, b)

                @pl.when(w + NBUF < NWIN)
                def _():
                    write_wait(w, b)
                    gather_start(w + NBUF, b)

        for b in range(NBUF):
            write_wait(NWIN - NBUF + b, b)

    return kern(xpad, idx_flat)


HW = CH // 2
_HIMASK = np.uint32(0xFFFF0000)


def _unpack(p):
    u = lax.bitcast_convert_type(p, jnp.uint32)
    lo = lax.bitcast_convert_type(u << 16, jnp.float32)
    hi = lax.bitcast_convert_type(u & _HIMASK, jnp.float32)
    return jnp.concatenate([lo, hi], axis=1).astype(jnp.bfloat16)


def _pack(x):
    b = x.astype(jnp.bfloat16)
    ulo = lax.bitcast_convert_type(
        b[:, :HW].astype(jnp.float32), jnp.uint32)
    uhi = lax.bitcast_convert_type(
        b[:, HW:].astype(jnp.float32), jnp.uint32)
    return lax.bitcast_convert_type((ulo >> 16) | (uhi & _HIMASK),
                                    jnp.float32)


def _tc_conv(g, w, b, a, res, out_shape, out_dtype, mask_tail, pack_out):
    residual = res is not None

    def body(*refs):
        if residual:
            g_ref, w_ref, b_ref, a_ref, res_ref, o_ref, acc_ref = refs
        else:
            g_ref, w_ref, b_ref, a_ref, o_ref, acc_ref = refs
        k = pl.program_id(1)

        @pl.when(k == 0)
        def _():
            acc_ref[...] = jnp.zeros_like(acc_ref)

        acc_ref[...] += jnp.dot(_unpack(g_ref[...]), w_ref[k],
                                preferred_element_type=jnp.float32)

        @pl.when(k == K - 1)
        def _():
            x = acc_ref[...] + b_ref[...]
            if residual:
                x = x + res_ref[...]
            x = jnp.where(x >= 0, x, a_ref[0, 0] * x)
            if mask_tail:
                nt = pl.program_id(0)
                rows = nt * TN + jax.lax.broadcasted_iota(
                    jnp.int32, (TN, CH), 0)
                x = jnp.where(rows < N, x, 0.0)
            if pack_out:
                o_ref[...] = _pack(x)
            else:
                o_ref[...] = x.astype(o_ref.dtype)

    in_specs = [
        pl.BlockSpec((TN, HW), lambda nt, k: (k * NT + nt, 0)),
        pl.BlockSpec((K, CH, CH), lambda nt, k: (0, 0, 0)),
        pl.BlockSpec((1, CH), lambda nt, k: (0, 0)),
        pl.BlockSpec((1, 1), lambda nt, k: (0, 0)),
    ]
    args = [g, w, b.reshape(1, CH), a.reshape(1, 1)]
    if residual:
        in_specs.append(pl.BlockSpec((TN, CH), lambda nt, k: (nt, 0)))
        args.append(res)

    ow = HW if pack_out else CH
    return pl.pallas_call(
        body,
        grid=(NT, K),
        in_specs=in_specs,
        out_specs=pl.BlockSpec((TN, ow), lambda nt, k: (nt, 0)),
        out_shape=jax.ShapeDtypeStruct(out_shape, out_dtype),
        scratch_shapes=[pltpu.VMEM((TN, CH), jnp.float32)],
        compiler_params=pltpu.CompilerParams(
            dimension_semantics=("parallel", "arbitrary")),
    )(*args)


def kernel(feats, nbr_idx, W1, b1, a1, W2, b2, a2):
    idxT = nbr_idx.T.astype(jnp.int32)
    idx_pad = jnp.full((K, NPAD), N, jnp.int32).at[:, :N].set(idxT)
    flat0 = jnp.full((L,), N, jnp.int32).at[:ROWS].set(
        idx_pad.reshape(-1))
    pos = jnp.arange(L, dtype=jnp.int32)
    idx_flat = jnp.where(flat0 == N, N + pos % (NPAD - N), flat0)

    w1b = W1.astype(jnp.bfloat16)
    w2b = W2.astype(jnp.bfloat16)

    fb = feats.astype(jnp.bfloat16)
    ulo = lax.bitcast_convert_type(
        fb[:, :HW].astype(jnp.float32), jnp.uint32)
    uhi = lax.bitcast_convert_type(
        fb[:, HW:].astype(jnp.float32), jnp.uint32)
    fpacked = lax.bitcast_convert_type((ulo >> 16) | (uhi & _HIMASK),
                                       jnp.float32)
    xpad0 = jnp.concatenate(
        [fpacked, jnp.zeros((NPAD - N, HW), jnp.float32)], axis=0)
    g1 = _sc_gather(xpad0, idx_flat)
    x1 = _tc_conv(g1, w1b, b1, a1, None, (NPAD, HW), jnp.float32,
                  mask_tail=True, pack_out=True)

    g2 = _sc_gather(x1, idx_flat)
    out = _tc_conv(g2, w2b, b2, a2, feats, (N, CH), jnp.float32,
                   mask_tail=False, pack_out=False)
    return out


# speedup vs baseline: 6.1493x; 1.1358x over previous
import functools

import numpy as np
import jax
import jax.numpy as jnp
from jax import lax
from jax.experimental import pallas as pl
from jax.experimental.pallas import tpu as pltpu
from jax.experimental.pallas import tpu_sc as plsc

N = 10000
CH = 256
K = 27
TN = 5120
NT = (N + TN - 1) // TN
NPAD = NT * TN
GW = 64
NWORK = 32
NBUF = 4
ROWS = K * NPAD
_CHUNK = GW * NWORK * NBUF
L = ((ROWS + _CHUNK - 1) // _CHUNK) * _CHUNK
PER = L // NWORK
NWIN = PER // GW
assert NWIN % NBUF == 0


def _sc_gather(xpad, idx_flat):
    width = xpad.shape[1]

    @functools.partial(
        pl.kernel,
        out_type=jax.ShapeDtypeStruct((L, width), xpad.dtype),
        mesh=plsc.VectorSubcoreMesh(core_axis_name="c", subcore_axis_name="s"),
        scratch_types=[
            pltpu.VMEM((PER,), jnp.int32),
            pltpu.VMEM((NBUF, GW, width), xpad.dtype),
            pltpu.SemaphoreType.DMA((NBUF,)),
            pltpu.SemaphoreType.DMA((NBUF,)),
            pltpu.SemaphoreType.DMA,
        ],
    )
    def kern(x_hbm, i_hbm, o_hbm, idx_v, rows_v, gsem, osem, isem):
        wid = lax.axis_index("s") * 2 + lax.axis_index("c")
        base = wid * PER
        pltpu.async_copy(i_hbm.at[pl.ds(base, PER)], idx_v, isem).wait()

        def gather_start(w, b):
            pltpu.make_async_copy(
                x_hbm.at[idx_v.at[pl.ds(w * GW, GW)]],
                rows_v.at[b], gsem.at[b]).start()

        def write_start(w, b):
            pltpu.make_async_copy(
                rows_v.at[b],
                o_hbm.at[pl.ds(base + w * GW, GW)], osem.at[b]).start()

        def gather_wait(b):
            pltpu.make_async_copy(
                x_hbm.at[idx_v.at[pl.ds(0, GW)]],
                rows_v.at[b], gsem.at[b]).wait()

        def write_wait(w, b):
            pltpu.make_async_copy(
                rows_v.at[b],
                o_hbm.at[pl.ds(base + w * GW, GW)], osem.at[b]).wait()

        for b in range(NBUF):
            gather_start(b, b)

        @pl.loop(0, NWIN, step=NBUF)
        def _(g):
            for b in range(NBUF):
                w = g + b
                gather_wait(b)
                write_start(w, b)

                @pl.when(w + NBUF < NWIN)
                def _():
                    write_wait(w, b)
                    gather_start(w + NBUF, b)

        for b in range(NBUF):
            write_wait(NWIN - NBUF + b, b)

    return kern(xpad, idx_flat)


HW = CH // 2
_HIMASK = np.uint32(0xFFFF0000)


def _unpack(p):
    u = lax.bitcast_convert_type(p, jnp.uint32)
    lo = lax.bitcast_convert_type(u << 16, jnp.float32)
    hi = lax.bitcast_convert_type(u & _HIMASK, jnp.float32)
    return jnp.concatenate([lo, hi], axis=1).astype(jnp.bfloat16)


def _pack(x):
    b = x.astype(jnp.bfloat16)
    ulo = lax.bitcast_convert_type(
        b[:, :HW].astype(jnp.float32), jnp.uint32)
    uhi = lax.bitcast_convert_type(
        b[:, HW:].astype(jnp.float32), jnp.uint32)
    return lax.bitcast_convert_type((ulo >> 16) | (uhi & _HIMASK),
                                    jnp.float32)


def _tc_conv(g, w, b, a, res, out_shape, out_dtype, mask_tail, pack_out):
    residual = res is not None

    def body(*refs):
        if residual:
            g_ref, w_ref, b_ref, a_ref, res_ref, o_ref, acc_ref = refs
        else:
            g_ref, w_ref, b_ref, a_ref, o_ref, acc_ref = refs
        k = pl.program_id(1)

        @pl.when(k == 0)
        def _():
            acc_ref[...] = jnp.zeros_like(acc_ref)

        acc_ref[...] += jnp.dot(_unpack(g_ref[...]), w_ref[k],
                                preferred_element_type=jnp.float32)

        @pl.when(k == K - 1)
        def _():
            x = acc_ref[...] + b_ref[...]
            if residual:
                x = x + res_ref[...]
            x = jnp.where(x >= 0, x, a_ref[0, 0] * x)
            if mask_tail:
                nt = pl.program_id(0)
                rows = nt * TN + jax.lax.broadcasted_iota(
                    jnp.int32, (TN, CH), 0)
                x = jnp.where(rows < N, x, 0.0)
            if pack_out:
                o_ref[...] = _pack(x)
            else:
                o_ref[...] = x.astype(o_ref.dtype)

    in_specs = [
        pl.BlockSpec((TN, HW), lambda nt, k: (k * NT + nt, 0)),
        pl.BlockSpec((K, CH, CH), lambda nt, k: (0, 0, 0)),
        pl.BlockSpec((1, CH), lambda nt, k: (0, 0)),
        pl.BlockSpec((1, 1), lambda nt, k: (0, 0)),
    ]
    args = [g, w, b.reshape(1, CH), a.reshape(1, 1)]
    if residual:
        in_specs.append(pl.BlockSpec((TN, CH), lambda nt, k: (nt, 0)))
        args.append(res)

    ow = HW if pack_out else CH
    return pl.pallas_call(
        body,
        grid=(NT, K),
        in_specs=in_specs,
        out_specs=pl.BlockSpec((TN, ow), lambda nt, k: (nt, 0)),
        out_shape=jax.ShapeDtypeStruct(out_shape, out_dtype),
        scratch_shapes=[pltpu.VMEM((TN, CH), jnp.float32)],
        compiler_params=pltpu.CompilerParams(
            dimension_semantics=("parallel", "arbitrary")),
    )(*args)


def kernel(feats, nbr_idx, W1, b1, a1, W2, b2, a2):
    idxT = nbr_idx.T.astype(jnp.int32)
    idx_pad = jnp.full((K, NPAD), N, jnp.int32).at[:, :N].set(idxT)
    flat0 = jnp.full((L,), N, jnp.int32).at[:ROWS].set(
        idx_pad.reshape(-1))
    pos = jnp.arange(L, dtype=jnp.int32)
    idx_flat = jnp.where(flat0 == N, N + pos % (NPAD - N), flat0)

    w1b = W1.astype(jnp.bfloat16)
    w2b = W2.astype(jnp.bfloat16)

    fb = feats.astype(jnp.bfloat16)
    ulo = lax.bitcast_convert_type(
        fb[:, :HW].astype(jnp.float32), jnp.uint32)
    uhi = lax.bitcast_convert_type(
        fb[:, HW:].astype(jnp.float32), jnp.uint32)
    fpacked = lax.bitcast_convert_type((ulo >> 16) | (uhi & _HIMASK),
                                       jnp.float32)
    xpad0 = jnp.concatenate(
        [fpacked, jnp.zeros((NPAD - N, HW), jnp.float32)], axis=0)
    g1 = _sc_gather(xpad0, idx_flat)
    x1 = _tc_conv(g1, w1b, b1, a1, None, (NPAD, HW), jnp.float32,
                  mask_tail=True, pack_out=True)

    g2 = _sc_gather(x1, idx_flat)
    out = _tc_conv(g2, w2b, b2, a2, feats, (N, CH), jnp.float32,
                   mask_tail=False, pack_out=False)
    return out


# speedup vs baseline: 6.1638x; 1.0024x over previous
import functools

import numpy as np
import jax
import jax.numpy as jnp
from jax import lax
from jax.experimental import pallas as pl
from jax.experimental.pallas import tpu as pltpu
from jax.experimental.pallas import tpu_sc as plsc

N = 10000
CH = 256
K = 27
TN = 5120
NT = (N + TN - 1) // TN
NPAD = NT * TN
GW = 128
NWORK = 32
NBUF = 2
ROWS = K * NPAD
_CHUNK = GW * NWORK * NBUF
L = ((ROWS + _CHUNK - 1) // _CHUNK) * _CHUNK
PER = L // NWORK
NWIN = PER // GW
assert NWIN % NBUF == 0


def _sc_gather(xpad, idx_flat):
    width = xpad.shape[1]

    @functools.partial(
        pl.kernel,
        out_type=jax.ShapeDtypeStruct((L, width), xpad.dtype),
        mesh=plsc.VectorSubcoreMesh(core_axis_name="c", subcore_axis_name="s"),
        scratch_types=[
            pltpu.VMEM((PER,), jnp.int32),
            pltpu.VMEM((NBUF, GW, width), xpad.dtype),
            pltpu.SemaphoreType.DMA((NBUF,)),
            pltpu.SemaphoreType.DMA((NBUF,)),
            pltpu.SemaphoreType.DMA,
        ],
    )
    def kern(x_hbm, i_hbm, o_hbm, idx_v, rows_v, gsem, osem, isem):
        wid = lax.axis_index("s") * 2 + lax.axis_index("c")
        base = wid * PER
        pltpu.async_copy(i_hbm.at[pl.ds(base, PER)], idx_v, isem).wait()

        def gather_start(w, b):
            pltpu.make_async_copy(
                x_hbm.at[idx_v.at[pl.ds(w * GW, GW)]],
                rows_v.at[b], gsem.at[b]).start()

        def write_start(w, b):
            pltpu.make_async_copy(
                rows_v.at[b],
                o_hbm.at[pl.ds(base + w * GW, GW)], osem.at[b]).start()

        def gather_wait(b):
            pltpu.make_async_copy(
                x_hbm.at[idx_v.at[pl.ds(0, GW)]],
                rows_v.at[b], gsem.at[b]).wait()

        def write_wait(w, b):
            pltpu.make_async_copy(
                rows_v.at[b],
                o_hbm.at[pl.ds(base + w * GW, GW)], osem.at[b]).wait()

        for b in range(NBUF):
            gather_start(b, b)

        @pl.loop(0, NWIN, step=NBUF)
        def _(g):
            for b in range(NBUF):
                w = g + b
                gather_wait(b)
                write_start(w, b)

                @pl.when(w + NBUF < NWIN)
                def _():
                    write_wait(w, b)
                    gather_start(w + NBUF, b)

        for b in range(NBUF):
            write_wait(NWIN - NBUF + b, b)

    return kern(xpad, idx_flat)


HW = CH // 2
_HIMASK = np.uint32(0xFFFF0000)


def _unpack(p):
    u = lax.bitcast_convert_type(p, jnp.uint32)
    lo = lax.bitcast_convert_type(u << 16, jnp.float32)
    hi = lax.bitcast_convert_type(u & _HIMASK, jnp.float32)
    return jnp.concatenate([lo, hi], axis=1).astype(jnp.bfloat16)


def _pack(x):
    b = x.astype(jnp.bfloat16)
    ulo = lax.bitcast_convert_type(
        b[:, :HW].astype(jnp.float32), jnp.uint32)
    uhi = lax.bitcast_convert_type(
        b[:, HW:].astype(jnp.float32), jnp.uint32)
    return lax.bitcast_convert_type((ulo >> 16) | (uhi & _HIMASK),
                                    jnp.float32)


def _tc_conv(g, w, b, a, res, out_shape, out_dtype, mask_tail, pack_out):
    residual = res is not None

    def body(*refs):
        if residual:
            g_ref, w_ref, b_ref, a_ref, res_ref, o_ref, acc_ref = refs
        else:
            g_ref, w_ref, b_ref, a_ref, o_ref, acc_ref = refs
        k = pl.program_id(1)

        @pl.when(k == 0)
        def _():
            acc_ref[...] = jnp.zeros_like(acc_ref)

        acc_ref[...] += jnp.dot(_unpack(g_ref[...]), w_ref[k],
                                preferred_element_type=jnp.float32)

        @pl.when(k == K - 1)
        def _():
            x = acc_ref[...] + b_ref[...]
            if residual:
                x = x + res_ref[...]
            x = jnp.where(x >= 0, x, a_ref[0, 0] * x)
            if mask_tail:
                nt = pl.program_id(0)
                rows = nt * TN + jax.lax.broadcasted_iota(
                    jnp.int32, (TN, CH), 0)
                x = jnp.where(rows < N, x, 0.0)
            if pack_out:
                o_ref[...] = _pack(x)
            else:
                o_ref[...] = x.astype(o_ref.dtype)

    in_specs = [
        pl.BlockSpec((TN, HW), lambda nt, k: (k * NT + nt, 0)),
        pl.BlockSpec((K, CH, CH), lambda nt, k: (0, 0, 0)),
        pl.BlockSpec((1, CH), lambda nt, k: (0, 0)),
        pl.BlockSpec((1, 1), lambda nt, k: (0, 0)),
    ]
    args = [g, w, b.reshape(1, CH), a.reshape(1, 1)]
    if residual:
        in_specs.append(pl.BlockSpec((TN, CH), lambda nt, k: (nt, 0)))
        args.append(res)

    ow = HW if pack_out else CH
    return pl.pallas_call(
        body,
        grid=(NT, K),
        in_specs=in_specs,
        out_specs=pl.BlockSpec((TN, ow), lambda nt, k: (nt, 0)),
        out_shape=jax.ShapeDtypeStruct(out_shape, out_dtype),
        scratch_shapes=[pltpu.VMEM((TN, CH), jnp.float32)],
        compiler_params=pltpu.CompilerParams(
            dimension_semantics=("parallel", "arbitrary")),
    )(*args)


def kernel(feats, nbr_idx, W1, b1, a1, W2, b2, a2):
    idxT = nbr_idx.T.astype(jnp.int32)
    idx_pad = jnp.full((K, NPAD), N, jnp.int32).at[:, :N].set(idxT)
    flat0 = jnp.full((L,), N, jnp.int32).at[:ROWS].set(
        idx_pad.reshape(-1))
    pos = jnp.arange(L, dtype=jnp.int32)
    idx_flat = jnp.where(flat0 == N, N + pos % (NPAD - N), flat0)

    w1b = W1.astype(jnp.bfloat16)
    w2b = W2.astype(jnp.bfloat16)

    fb = feats.astype(jnp.bfloat16)
    ulo = lax.bitcast_convert_type(
        fb[:, :HW].astype(jnp.float32), jnp.uint32)
    uhi = lax.bitcast_convert_type(
        fb[:, HW:].astype(jnp.float32), jnp.uint32)
    fpacked = lax.bitcast_convert_type((ulo >> 16) | (uhi & _HIMASK),
                                       jnp.float32)
    xpad0 = jnp.concatenate(
        [fpacked, jnp.zeros((NPAD - N, HW), jnp.float32)], axis=0)
    g1 = _sc_gather(xpad0, idx_flat)
    x1 = _tc_conv(g1, w1b, b1, a1, None, (NPAD, HW), jnp.float32,
                  mask_tail=True, pack_out=True)

    g2 = _sc_gather(x1, idx_flat)
    out = _tc_conv(g2, w2b, b2, a2, feats, (N, CH), jnp.float32,
                   mask_tail=False, pack_out=False)
    return out
